# baseline (speedup 1.0000x reference)
import functools
import jax
import jax.numpy as jnp
from jax import lax
from jax.experimental import pallas as pl
from jax.experimental.pallas import tpu as pltpu
from jax.experimental.pallas import tpu_sc as plsc

L = 16


def kernel(x, sym_table, sc_table):
    B, T, S, D = x.shape
    ROW = S * D
    NW = 32
    t_per_w = T // NW

    x2 = x.reshape(B * T, ROW)
    sc_flat = sc_table.reshape(ROW)
    mesh = plsc.VectorSubcoreMesh(core_axis_name="c", subcore_axis_name="s")

    @functools.partial(
        pl.kernel,
        mesh=mesh,
        out_type=jax.ShapeDtypeStruct((B * T, ROW), jnp.float32),
        scratch_types=[
            pltpu.VMEM((ROW,), jnp.float32),
            pltpu.VMEM((D,), jnp.float32),
            pltpu.VMEM((ROW,), jnp.float32),
            pltpu.VMEM((ROW,), jnp.float32),
            pltpu.SemaphoreType.DMA,
            pltpu.SemaphoreType.DMA,
        ],
    )
    def k(x_hbm, sym_hbm, sc_hbm, out_hbm, pos_v, sym_v, xa, xb, sema, semb):
        wid = lax.axis_index("s") * 2 + lax.axis_index("c")
        bufs = (xa, xb)
        sems = (sema, semb)
        for j in range(t_per_w):
            t = wid * t_per_w + j
            pltpu.sync_copy(sc_hbm, pos_v)
            pltpu.sync_copy(sym_hbm.at[t], sym_v)

            def pos_body(i, _):
                o = i * L
                od = lax.rem(i, D // L) * L
                pos_v[pl.ds(o, L)] = pos_v[pl.ds(o, L)] + sym_v[pl.ds(od, L)]
                return 0

            lax.fori_loop(0, ROW // L, pos_body, 0, unroll=4)

            handles = [pltpu.async_copy(x_hbm.at[t], bufs[0], sems[0]), None]
            for b in range(B):
                if b + 1 < B:
                    handles[(b + 1) % 2] = pltpu.async_copy(
                        x_hbm.at[t + (b + 1) * T], bufs[(b + 1) % 2],
                        sems[(b + 1) % 2])
                handles[b % 2].wait()
                buf = bufs[b % 2]

                def add_body(i, _, buf=buf):
                    o = i * L
                    buf[pl.ds(o, L)] = buf[pl.ds(o, L)] + pos_v[pl.ds(o, L)]
                    return 0

                lax.fori_loop(0, ROW // L, add_body, 0, unroll=4)
                pltpu.sync_copy(buf, out_hbm.at[t + b * T])

    out = k(x2, sym_table, sc_flat)
    return out.reshape(B, T, S, D)


# speedup vs baseline: 7.0330x; 7.0330x over previous
import jax
import jax.numpy as jnp
from jax.experimental import pallas as pl
from jax.experimental.pallas import tpu as pltpu


def _body(x_ref, sym_ref, sc_ref, o_ref):
    sym = sym_ref[...]
    sc = sc_ref[...]
    o_ref[...] = x_ref[...] + sym[:, None, :] + sc[None, :, :]


def kernel(x, sym_table, sc_table):
    B, T, S, D = x.shape
    BT = 64

    x3 = x.reshape(B * T, S, D)
    grid = (B * T // BT,)
    out = pl.pallas_call(
        _body,
        grid=grid,
        in_specs=[
            pl.BlockSpec((BT, S, D), lambda r: (r, 0, 0)),
            pl.BlockSpec((BT, D), lambda r: (r % (T // BT), 0)),
            pl.BlockSpec((S, D), lambda r: (0, 0)),
        ],
        out_specs=pl.BlockSpec((BT, S, D), lambda r: (r, 0, 0)),
        out_shape=jax.ShapeDtypeStruct((B * T, S, D), x.dtype),
        compiler_params=pltpu.CompilerParams(
            dimension_semantics=("arbitrary",),
        ),
    )(x3, sym_table, sc_table)
    return out.reshape(B, T, S, D)
